# baseline (speedup 1.0000x reference)
import jax
import jax.numpy as jnp
from jax.experimental import pallas as pl
from jax.experimental.pallas import tpu as pltpu

D_MODEL = 2048
SEQ = 8192
ROWS = 1024
GRID = SEQ // ROWS


def _body(params_ref, cycle_ref, strength_ref, out_ref):
    max_cycle = cycle_ref.shape[0]
    n_str = strength_ref.shape[0]
    width = max_cycle + n_str
    base = pl.program_id(0) * ROWS
    cyc = params_ref[0]
    taal = params_ref[1]
    i2 = base + jax.lax.broadcasted_iota(jnp.int32, (ROWS, width), 0)
    col = jax.lax.broadcasted_iota(jnp.int32, (ROWS, width), 1)
    pos = jax.lax.rem(i2, cyc)
    strength = jnp.where(jax.lax.rem(i2, taal) == 0, 0, 3) + max_cycle
    onehot = jnp.logical_or(col == pos, col == strength)
    table = jnp.concatenate([cycle_ref[...], strength_ref[...]], axis=0)
    out_ref[...] = jnp.dot(onehot.astype(jnp.float32), table,
                           preferred_element_type=jnp.float32)


def kernel(cycle_emb, strength_emb, seq_len, taal_cycle_len):
    max_cycle = cycle_emb.shape[0]
    taal = jnp.asarray(taal_cycle_len, jnp.int32)
    cyc = jnp.minimum(taal, jnp.int32(max_cycle))
    params = jnp.stack([cyc, taal])
    out = pl.pallas_call(
        _body,
        grid=(GRID,),
        in_specs=[
            pl.BlockSpec(memory_space=pltpu.SMEM),
            pl.BlockSpec((max_cycle, D_MODEL), lambda i: (0, 0)),
            pl.BlockSpec((strength_emb.shape[0], D_MODEL), lambda i: (0, 0)),
        ],
        out_specs=pl.BlockSpec((ROWS, D_MODEL), lambda i: (i, 0)),
        out_shape=jax.ShapeDtypeStruct((SEQ, D_MODEL), jnp.float32),
    )(params, cycle_emb, strength_emb)
    return out[None, ...]


# speedup vs baseline: 3.7429x; 3.7429x over previous
import jax
import jax.numpy as jnp
from jax.experimental import pallas as pl
from jax.experimental.pallas import tpu as pltpu

D_MODEL = 2048
SEQ = 8192
ROWS = 1024
GRID = SEQ // ROWS


def _body(params_ref, cycle_ref, strength_ref, out_ref):
    max_cycle = cycle_ref.shape[0]
    n_str = strength_ref.shape[0]
    width = max_cycle + n_str
    base = pl.program_id(0) * ROWS
    cyc = params_ref[0]
    taal = params_ref[1]
    i2 = base + jax.lax.broadcasted_iota(jnp.int32, (ROWS, width), 0)
    col = jax.lax.broadcasted_iota(jnp.int32, (ROWS, width), 1)
    i2f = i2.astype(jnp.float32)

    def frem(d):
        df = d.astype(jnp.float32)
        qf = jnp.floor(i2f * (jnp.float32(1.0) / df))
        r = i2 - (qf * df).astype(jnp.int32)
        r = jnp.where(r < 0, r + d, r)
        return jnp.where(r >= d, r - d, r)

    pos = frem(cyc)
    strength = jnp.where(frem(taal) == 0, max_cycle, max_cycle + 3)
    onehot = jnp.logical_or(col == pos, col == strength)
    table = jnp.concatenate([cycle_ref[...], strength_ref[...]], axis=0)
    out_ref[...] = jnp.dot(onehot.astype(jnp.float32), table,
                           preferred_element_type=jnp.float32)


def kernel(cycle_emb, strength_emb, seq_len, taal_cycle_len):
    max_cycle = cycle_emb.shape[0]
    taal = jnp.asarray(taal_cycle_len, jnp.int32)
    cyc = jnp.minimum(taal, jnp.int32(max_cycle))
    params = jnp.stack([cyc, taal])
    out = pl.pallas_call(
        _body,
        grid=(GRID,),
        in_specs=[
            pl.BlockSpec(memory_space=pltpu.SMEM),
            pl.BlockSpec((max_cycle, D_MODEL), lambda i: (0, 0)),
            pl.BlockSpec((strength_emb.shape[0], D_MODEL), lambda i: (0, 0)),
        ],
        out_specs=pl.BlockSpec((ROWS, D_MODEL), lambda i: (i, 0)),
        out_shape=jax.ShapeDtypeStruct((SEQ, D_MODEL), jnp.float32),
    )(params, cycle_emb, strength_emb)
    return out[None, ...]
